# Initial kernel scaffold; baseline (speedup 1.0000x reference)
import jax
import jax.numpy as jnp
from jax.experimental import pallas as pl


def kernel(x, edge_index, rev_edge_index, edge_attr, W_i, b_i, W_h, b_h, W_o, b_o):
    raise NotImplementedError("write your pallas kernel here")


# speedup vs baseline: 1.7742x; 1.7742x over previous
import functools

import jax
import jax.numpy as jnp
from jax import lax
from jax.experimental import pallas as pl
from jax.experimental.pallas import tpu as pltpu
from jax.experimental.pallas import tpu_sc as plsc

_DEPTH = 3
_NC = 2
_NS = 16
_NW = _NC * _NS
_CW = 80


def _row_block(bs, d):
    return pl.BlockSpec((bs, d), lambda i: (i, 0))


def _rep_block(shape):
    return pl.BlockSpec(shape, lambda i: tuple(0 for _ in shape))


def _sc_gather(table, idx):
    t, d = table.shape
    e = idx.shape[0]
    per_w = e // _NW
    n_ch = per_w // _CW
    mesh = plsc.VectorSubcoreMesh(core_axis_name="c", subcore_axis_name="s")

    @functools.partial(
        pl.kernel,
        out_type=jax.ShapeDtypeStruct((e, d), jnp.float32),
        mesh=mesh,
        scratch_types=[
            pltpu.VMEM((_CW,), jnp.int32),
            pltpu.VMEM((_CW, d), jnp.float32),
            pltpu.SemaphoreType.DMA,
        ],
    )
    def gk(table_hbm, idx_hbm, out_hbm, idx_v, rows_v, sem):
        wid = lax.axis_index("s") * _NC + lax.axis_index("c")

        def body(i, carry):
            base = wid * per_w + i * _CW
            pltpu.sync_copy(idx_hbm.at[pl.ds(base, _CW)], idx_v)
            pltpu.async_copy(table_hbm.at[idx_v], rows_v, sem).wait()
            pltpu.sync_copy(rows_v, out_hbm.at[pl.ds(base, _CW)])
            return carry

        lax.fori_loop(0, n_ch, body, 0)

    return gk(table, idx)


def _sc_scatter_add(vals, idx, zeros_nd):
    e, d = vals.shape
    n = zeros_nd.shape[0]
    per_w = e // _NW
    n_ch = per_w // _CW
    rpt = 640
    tail = n - (_NS - 1) * rpt
    mesh = plsc.VectorSubcoreMesh(core_axis_name="c", subcore_axis_name="s")

    @functools.partial(
        pl.kernel,
        out_type=jax.ShapeDtypeStruct((_NC, n, d), jnp.float32),
        mesh=mesh,
        scratch_types=[
            pltpu.VMEM((_CW,), jnp.int32),
            pltpu.VMEM((_CW, d), jnp.float32),
            pltpu.VMEM_SHARED((n, d), jnp.float32),
            pltpu.SemaphoreType.DMA,
        ],
    )
    def sk(vals_hbm, idx_hbm, zeros_hbm, out_hbm, idx_v, rows_v, acc_sh, sem):
        cid = lax.axis_index("c")
        sid = lax.axis_index("s")
        wid = sid * _NC + cid
        @pl.when(sid < _NS - 1)
        def _():
            pltpu.sync_copy(zeros_hbm.at[pl.ds(sid * rpt, rpt)],
                            acc_sh.at[pl.ds(sid * rpt, rpt)])

        @pl.when(sid == _NS - 1)
        def _():
            pltpu.sync_copy(zeros_hbm.at[pl.ds((_NS - 1) * rpt, tail)],
                            acc_sh.at[pl.ds((_NS - 1) * rpt, tail)])

        plsc.subcore_barrier()

        def body(i, carry):
            base = wid * per_w + i * _CW
            pltpu.sync_copy(idx_hbm.at[pl.ds(base, _CW)], idx_v)
            pltpu.sync_copy(vals_hbm.at[pl.ds(base, _CW)], rows_v)
            pltpu.sync_copy(rows_v, acc_sh.at[idx_v], add=True)
            return carry

        lax.fori_loop(0, n_ch, body, 0)
        plsc.subcore_barrier()

        @pl.when(sid < _NS - 1)
        def _():
            pltpu.sync_copy(acc_sh.at[pl.ds(sid * rpt, rpt)],
                            out_hbm.at[cid, pl.ds(sid * rpt, rpt)])

        @pl.when(sid == _NS - 1)
        def _():
            pltpu.sync_copy(acc_sh.at[pl.ds((_NS - 1) * rpt, tail)],
                            out_hbm.at[cid, pl.ds((_NS - 1) * rpt, tail)])

    return sk(vals, idx, zeros_nd)


def _tc_matmul(a, w):
    m, k = a.shape
    n = w.shape[1]
    bs = 2000

    def body(a_ref, w_ref, o_ref):
        o_ref[...] = jnp.dot(a_ref[...], w_ref[...],
                             preferred_element_type=jnp.float32)

    return pl.pallas_call(
        body,
        grid=(m // bs,),
        in_specs=[_row_block(bs, k), _rep_block((k, n))],
        out_specs=_row_block(bs, n),
        out_shape=jax.ShapeDtypeStruct((m, n), jnp.float32),
    )(a, w)


def _tc_init(g, ea, wie, bi, wh):
    e, h = g.shape
    de = ea.shape[1]
    bs = 2560

    def body(g_ref, ea_ref, wie_ref, bi_ref, wh_ref, h0_ref, p_ref):
        h0 = (g_ref[...]
              + jnp.dot(ea_ref[...], wie_ref[...],
                        preferred_element_type=jnp.float32)
              + bi_ref[...])
        h0_ref[...] = h0
        p_ref[...] = jnp.dot(jnp.maximum(h0, 0.0), wh_ref[...],
                             preferred_element_type=jnp.float32)

    return pl.pallas_call(
        body,
        grid=(e // bs,),
        in_specs=[_row_block(bs, h), _row_block(bs, de), _rep_block((de, h)),
                  _rep_block((1, h)), _rep_block((h, h))],
        out_specs=[_row_block(bs, h), _row_block(bs, h)],
        out_shape=[jax.ShapeDtypeStruct((e, h), jnp.float32),
                   jax.ShapeDtypeStruct((e, h), jnp.float32)],
    )(g, ea, wie, bi, wh)


def _tc_step(h0, a, b, bh, wh):
    e, h = h0.shape
    bs = 2560

    def body(h0_ref, a_ref, b_ref, bh_ref, wh_ref, p_ref):
        ht = jnp.maximum(h0_ref[...] + a_ref[...] - b_ref[...] + bh_ref[...],
                         0.0)
        p_ref[...] = jnp.dot(ht, wh_ref[...],
                             preferred_element_type=jnp.float32)

    return pl.pallas_call(
        body,
        grid=(e // bs,),
        in_specs=[_row_block(bs, h)] * 3 + [_rep_block((1, h)),
                                            _rep_block((h, h))],
        out_specs=_row_block(bs, h),
        out_shape=jax.ShapeDtypeStruct((e, h), jnp.float32),
    )(h0, a, b, bh, wh)


def _tc_last(h0, a, b, bh):
    e, h = h0.shape
    bs = 2560

    def body(h0_ref, a_ref, b_ref, bh_ref, o_ref):
        o_ref[...] = jnp.maximum(
            h0_ref[...] + a_ref[...] - b_ref[...] + bh_ref[...], 0.0)

    return pl.pallas_call(
        body,
        grid=(e // bs,),
        in_specs=[_row_block(bs, h)] * 3 + [_rep_block((1, h))],
        out_specs=_row_block(bs, h),
        out_shape=jax.ShapeDtypeStruct((e, h), jnp.float32),
    )(h0, a, b, bh)


def _tc_merge(p0, p1):
    n, h = p0.shape
    bs = 2000

    def body(a_ref, b_ref, o_ref):
        o_ref[...] = a_ref[...] + b_ref[...]

    return pl.pallas_call(
        body,
        grid=(n // bs,),
        in_specs=[_row_block(bs, h)] * 2,
        out_specs=_row_block(bs, h),
        out_shape=jax.ShapeDtypeStruct((n, h), jnp.float32),
    )(p0, p1)


def _tc_final(p0, p1, x, wox, wom, bo):
    n, h = x.shape
    bs = 2000

    def body(p0_ref, p1_ref, x_ref, wox_ref, wom_ref, bo_ref, o_ref):
        ns = p0_ref[...] + p1_ref[...]
        s = jnp.sum(ns, axis=1, keepdims=True)
        m = jnp.where(s == 0.0, x_ref[...], ns)
        o_ref[...] = jnp.maximum(
            jnp.dot(x_ref[...], wox_ref[...],
                    preferred_element_type=jnp.float32)
            + jnp.dot(m, wom_ref[...], preferred_element_type=jnp.float32)
            + bo_ref[...], 0.0)

    return pl.pallas_call(
        body,
        grid=(n // bs,),
        in_specs=[_row_block(bs, h)] * 3 + [_rep_block((h, h))] * 2
                 + [_rep_block((1, h))],
        out_specs=_row_block(bs, h),
        out_shape=jax.ShapeDtypeStruct((n, h), jnp.float32),
    )(p0, p1, x, wox, wom, bo)


def kernel(x, edge_index, rev_edge_index, edge_attr, W_i, b_i, W_h, b_h,
           W_o, b_o):
    n, df = x.shape
    h = W_h.shape[0]
    src = edge_index[0]
    dst = edge_index[1]
    wi_x, wi_e = W_i[:df], W_i[df:]
    wo_x, wo_m = W_o[:df], W_o[df:]
    bi = b_i.reshape(1, h)
    bh = b_h.reshape(1, h)
    bo = b_o.reshape(1, h)
    zeros_nd = jnp.zeros((n, h), jnp.float32)

    hx = _tc_matmul(x, wi_x)
    g0 = _sc_gather(hx, src)
    h0, p = _tc_init(g0, edge_attr, wi_e, bi, W_h)

    for _ in range(_DEPTH - 2):
        parts = _sc_scatter_add(p, dst, zeros_nd)
        ns = _tc_merge(parts[0], parts[1])
        a = _sc_gather(ns, src)
        b = _sc_gather(p, rev_edge_index)
        p = _tc_step(h0, a, b, bh, W_h)

    parts = _sc_scatter_add(p, dst, zeros_nd)
    ns = _tc_merge(parts[0], parts[1])
    a = _sc_gather(ns, src)
    b = _sc_gather(p, rev_edge_index)
    h_fin = _tc_last(h0, a, b, bh)

    parts = _sc_scatter_add(h_fin, dst, zeros_nd)
    return _tc_final(parts[0], parts[1], x, wo_x, wo_m, bo)


# speedup vs baseline: 2.9573x; 1.6669x over previous
import functools

import jax
import jax.numpy as jnp
from jax import lax
from jax.experimental import pallas as pl
from jax.experimental.pallas import tpu as pltpu
from jax.experimental.pallas import tpu_sc as plsc

_DEPTH = 3
_NC = 2
_NS = 16
_NW = _NC * _NS
_CW = 80


def _row_block(bs, d):
    return pl.BlockSpec((bs, d), lambda i: (i, 0))


def _rep_block(shape):
    return pl.BlockSpec(shape, lambda i: tuple(0 for _ in shape))


_NBUF = 5


def _sc_gather(table, idx):
    t, d = table.shape
    e = idx.shape[0]
    per_w = e // _NW
    n_ch = per_w // _CW
    n_gr = n_ch // _NBUF
    mesh = plsc.VectorSubcoreMesh(core_axis_name="c", subcore_axis_name="s")

    @functools.partial(
        pl.kernel,
        out_type=jax.ShapeDtypeStruct((e, d), jnp.float32),
        mesh=mesh,
        scratch_types=[
            pltpu.VMEM((_NBUF, _CW), jnp.int32),
            pltpu.VMEM((_NBUF, _CW, d), jnp.float32),
            pltpu.SemaphoreType.DMA((_NBUF,)),
            pltpu.SemaphoreType.DMA((_NBUF,)),
            pltpu.SemaphoreType.DMA((_NBUF,)),
        ],
    )
    def gk(table_hbm, idx_hbm, out_hbm, idx_v, rows_v, isem, gsem, ssem):
        wid = lax.axis_index("s") * _NC + lax.axis_index("c")
        base_w = wid * per_w

        def idx_copy(c, b):
            return pltpu.make_async_copy(
                idx_hbm.at[pl.ds(base_w + c * _CW, _CW)], idx_v.at[b],
                isem.at[b])

        def out_copy(c, b):
            return pltpu.make_async_copy(
                rows_v.at[b], out_hbm.at[pl.ds(base_w + c * _CW, _CW)],
                ssem.at[b])

        for b in range(_NBUF):
            idx_copy(b, b).start()

        def group(g, carry):
            c0 = g * _NBUF
            gathers = []
            for b in range(_NBUF):
                @pl.when(g > 0)
                def _():
                    out_copy(0, b).wait()

                idx_copy(c0 + b, b).wait()
                gathers.append(pltpu.async_copy(
                    table_hbm.at[idx_v.at[b]], rows_v.at[b], gsem.at[b]))
            for b in range(_NBUF):
                gathers[b].wait()
                out_copy(c0 + b, b).start()

                @pl.when(g + 1 < n_gr)
                def _():
                    idx_copy(c0 + _NBUF + b, b).start()
            return carry

        lax.fori_loop(0, n_gr, group, 0)
        for b in range(_NBUF):
            out_copy(0, b).wait()

    return gk(table, idx)


def _sc_scatter_add(vals, idx, zeros_nd):
    e, d = vals.shape
    n = zeros_nd.shape[0]
    per_w = e // _NW
    cw = 40
    n_ch = per_w // cw
    rpt = 640
    tail = n - (_NS - 1) * rpt
    mesh = plsc.VectorSubcoreMesh(core_axis_name="c", subcore_axis_name="s")

    n_gr = n_ch // _NBUF

    @functools.partial(
        pl.kernel,
        out_type=jax.ShapeDtypeStruct((_NC, n, d), jnp.float32),
        mesh=mesh,
        scratch_types=[
            pltpu.VMEM((_NBUF, cw), jnp.int32),
            pltpu.VMEM((_NBUF, cw, d), jnp.float32),
            pltpu.VMEM_SHARED((n, d), jnp.float32),
            pltpu.SemaphoreType.DMA((_NBUF,)),
            pltpu.SemaphoreType.DMA((_NBUF,)),
            pltpu.SemaphoreType.DMA((_NBUF,)),
        ],
    )
    def sk(vals_hbm, idx_hbm, zeros_hbm, out_hbm, idx_v, rows_v, acc_sh,
           isem, vsem, ssem):
        cid = lax.axis_index("c")
        sid = lax.axis_index("s")
        wid = sid * _NC + cid
        base_w = wid * per_w

        @pl.when(sid < _NS - 1)
        def _():
            pltpu.sync_copy(zeros_hbm.at[pl.ds(sid * rpt, rpt)],
                            acc_sh.at[pl.ds(sid * rpt, rpt)])

        @pl.when(sid == _NS - 1)
        def _():
            pltpu.sync_copy(zeros_hbm.at[pl.ds((_NS - 1) * rpt, tail)],
                            acc_sh.at[pl.ds((_NS - 1) * rpt, tail)])

        plsc.subcore_barrier()

        def idx_copy(c, b):
            return pltpu.make_async_copy(
                idx_hbm.at[pl.ds(base_w + c * cw, cw)], idx_v.at[b],
                isem.at[b])

        def val_copy(c, b):
            return pltpu.make_async_copy(
                vals_hbm.at[pl.ds(base_w + c * cw, cw)], rows_v.at[b],
                vsem.at[b])

        for b in range(_NBUF):
            idx_copy(b, b).start()
            val_copy(b, b).start()

        def group(g, carry):
            c0 = g * _NBUF
            scat = []
            for b in range(_NBUF):
                idx_copy(c0 + b, b).wait()
                val_copy(c0 + b, b).wait()
                scat.append(pltpu.async_copy(
                    rows_v.at[b], acc_sh.at[idx_v.at[b]], ssem.at[b],
                    add=True))
            for b in range(_NBUF):
                scat[b].wait()

                @pl.when(g + 1 < n_gr)
                def _():
                    idx_copy(c0 + _NBUF + b, b).start()
                    val_copy(c0 + _NBUF + b, b).start()
            return carry

        lax.fori_loop(0, n_gr, group, 0)
        plsc.subcore_barrier()

        @pl.when(sid < _NS - 1)
        def _():
            pltpu.sync_copy(acc_sh.at[pl.ds(sid * rpt, rpt)],
                            out_hbm.at[cid, pl.ds(sid * rpt, rpt)])

        @pl.when(sid == _NS - 1)
        def _():
            pltpu.sync_copy(acc_sh.at[pl.ds((_NS - 1) * rpt, tail)],
                            out_hbm.at[cid, pl.ds((_NS - 1) * rpt, tail)])

    return sk(vals, idx, zeros_nd)


def _tc_matmul(a, w):
    m, k = a.shape
    n = w.shape[1]
    bs = 2000

    def body(a_ref, w_ref, o_ref):
        o_ref[...] = jnp.dot(a_ref[...], w_ref[...],
                             preferred_element_type=jnp.float32)

    return pl.pallas_call(
        body,
        grid=(m // bs,),
        in_specs=[_row_block(bs, k), _rep_block((k, n))],
        out_specs=_row_block(bs, n),
        out_shape=jax.ShapeDtypeStruct((m, n), jnp.float32),
    )(a, w)


def _tc_init(g, ea, wie, bi, wh):
    e, h = g.shape
    de = ea.shape[1]
    bs = 2560

    def body(g_ref, ea_ref, wie_ref, bi_ref, wh_ref, h0_ref, p_ref):
        h0 = (g_ref[...]
              + jnp.dot(ea_ref[...], wie_ref[...],
                        preferred_element_type=jnp.float32)
              + bi_ref[...])
        h0_ref[...] = h0
        p_ref[...] = jnp.dot(jnp.maximum(h0, 0.0), wh_ref[...],
                             preferred_element_type=jnp.float32)

    return pl.pallas_call(
        body,
        grid=(e // bs,),
        in_specs=[_row_block(bs, h), _row_block(bs, de), _rep_block((de, h)),
                  _rep_block((1, h)), _rep_block((h, h))],
        out_specs=[_row_block(bs, h), _row_block(bs, h)],
        out_shape=[jax.ShapeDtypeStruct((e, h), jnp.float32),
                   jax.ShapeDtypeStruct((e, h), jnp.float32)],
    )(g, ea, wie, bi, wh)


def _tc_step(h0, a, b, bh, wh):
    e, h = h0.shape
    bs = 2560

    def body(h0_ref, a_ref, b_ref, bh_ref, wh_ref, p_ref):
        ht = jnp.maximum(h0_ref[...] + a_ref[...] - b_ref[...] + bh_ref[...],
                         0.0)
        p_ref[...] = jnp.dot(ht, wh_ref[...],
                             preferred_element_type=jnp.float32)

    return pl.pallas_call(
        body,
        grid=(e // bs,),
        in_specs=[_row_block(bs, h)] * 3 + [_rep_block((1, h)),
                                            _rep_block((h, h))],
        out_specs=_row_block(bs, h),
        out_shape=jax.ShapeDtypeStruct((e, h), jnp.float32),
    )(h0, a, b, bh, wh)


def _tc_last(h0, a, b, bh):
    e, h = h0.shape
    bs = 2560

    def body(h0_ref, a_ref, b_ref, bh_ref, o_ref):
        o_ref[...] = jnp.maximum(
            h0_ref[...] + a_ref[...] - b_ref[...] + bh_ref[...], 0.0)

    return pl.pallas_call(
        body,
        grid=(e // bs,),
        in_specs=[_row_block(bs, h)] * 3 + [_rep_block((1, h))],
        out_specs=_row_block(bs, h),
        out_shape=jax.ShapeDtypeStruct((e, h), jnp.float32),
    )(h0, a, b, bh)


def _tc_merge(p0, p1):
    n, h = p0.shape
    bs = 2000

    def body(a_ref, b_ref, o_ref):
        o_ref[...] = a_ref[...] + b_ref[...]

    return pl.pallas_call(
        body,
        grid=(n // bs,),
        in_specs=[_row_block(bs, h)] * 2,
        out_specs=_row_block(bs, h),
        out_shape=jax.ShapeDtypeStruct((n, h), jnp.float32),
    )(p0, p1)


def _tc_final(p0, p1, x, wox, wom, bo):
    n, h = x.shape
    bs = 2000

    def body(p0_ref, p1_ref, x_ref, wox_ref, wom_ref, bo_ref, o_ref):
        ns = p0_ref[...] + p1_ref[...]
        s = jnp.sum(ns, axis=1, keepdims=True)
        m = jnp.where(s == 0.0, x_ref[...], ns)
        o_ref[...] = jnp.maximum(
            jnp.dot(x_ref[...], wox_ref[...],
                    preferred_element_type=jnp.float32)
            + jnp.dot(m, wom_ref[...], preferred_element_type=jnp.float32)
            + bo_ref[...], 0.0)

    return pl.pallas_call(
        body,
        grid=(n // bs,),
        in_specs=[_row_block(bs, h)] * 3 + [_rep_block((h, h))] * 2
                 + [_rep_block((1, h))],
        out_specs=_row_block(bs, h),
        out_shape=jax.ShapeDtypeStruct((n, h), jnp.float32),
    )(p0, p1, x, wox, wom, bo)


def kernel(x, edge_index, rev_edge_index, edge_attr, W_i, b_i, W_h, b_h,
           W_o, b_o):
    n, df = x.shape
    h = W_h.shape[0]
    src = edge_index[0]
    dst = edge_index[1]
    wi_x, wi_e = W_i[:df], W_i[df:]
    wo_x, wo_m = W_o[:df], W_o[df:]
    bi = b_i.reshape(1, h)
    bh = b_h.reshape(1, h)
    bo = b_o.reshape(1, h)
    zeros_nd = jnp.zeros((n, h), jnp.float32)

    hx = _tc_matmul(x, wi_x)
    g0 = _sc_gather(hx, src)
    h0, p = _tc_init(g0, edge_attr, wi_e, bi, W_h)

    for _ in range(_DEPTH - 2):
        parts = _sc_scatter_add(p, dst, zeros_nd)
        ns = _tc_merge(parts[0], parts[1])
        a = _sc_gather(ns, src)
        b = _sc_gather(p, rev_edge_index)
        p = _tc_step(h0, a, b, bh, W_h)

    parts = _sc_scatter_add(p, dst, zeros_nd)
    ns = _tc_merge(parts[0], parts[1])
    a = _sc_gather(ns, src)
    b = _sc_gather(p, rev_edge_index)
    h_fin = _tc_last(h0, a, b, bh)

    parts = _sc_scatter_add(h_fin, dst, zeros_nd)
    return _tc_final(parts[0], parts[1], x, wo_x, wo_m, bo)


# speedup vs baseline: 2.9831x; 1.0087x over previous
import functools

import jax
import jax.numpy as jnp
from jax import lax
from jax.experimental import pallas as pl
from jax.experimental.pallas import tpu as pltpu
from jax.experimental.pallas import tpu_sc as plsc

_DEPTH = 3
_NC = 2
_NS = 16
_NW = _NC * _NS
_CW = 80


def _row_block(bs, d):
    return pl.BlockSpec((bs, d), lambda i: (i, 0))


def _rep_block(shape):
    return pl.BlockSpec(shape, lambda i: tuple(0 for _ in shape))


_NBUF = 5


def _sc_gather(table, idx):
    t, d = table.shape
    e = idx.shape[0]
    per_w = e // _NW
    n_ch = per_w // _CW
    n_gr = n_ch // _NBUF
    mesh = plsc.VectorSubcoreMesh(core_axis_name="c", subcore_axis_name="s")

    @functools.partial(
        pl.kernel,
        out_type=jax.ShapeDtypeStruct((e, d), jnp.float32),
        mesh=mesh,
        scratch_types=[
            pltpu.VMEM((_NBUF, _CW), jnp.int32),
            pltpu.VMEM((_NBUF, _CW, d), jnp.float32),
            pltpu.SemaphoreType.DMA((_NBUF,)),
            pltpu.SemaphoreType.DMA((_NBUF,)),
            pltpu.SemaphoreType.DMA((_NBUF,)),
        ],
    )
    def gk(table_hbm, idx_hbm, out_hbm, idx_v, rows_v, isem, gsem, ssem):
        wid = lax.axis_index("s") * _NC + lax.axis_index("c")
        base_w = wid * per_w

        def idx_copy(c, b):
            return pltpu.make_async_copy(
                idx_hbm.at[pl.ds(base_w + c * _CW, _CW)], idx_v.at[b],
                isem.at[b])

        def out_copy(c, b):
            return pltpu.make_async_copy(
                rows_v.at[b], out_hbm.at[pl.ds(base_w + c * _CW, _CW)],
                ssem.at[b])

        for b in range(_NBUF):
            idx_copy(b, b).start()

        def group(g, carry):
            c0 = g * _NBUF
            gathers = []
            for b in range(_NBUF):
                @pl.when(g > 0)
                def _():
                    out_copy(0, b).wait()

                idx_copy(c0 + b, b).wait()
                gathers.append(pltpu.async_copy(
                    table_hbm.at[idx_v.at[b]], rows_v.at[b], gsem.at[b]))
            for b in range(_NBUF):
                gathers[b].wait()
                out_copy(c0 + b, b).start()

                @pl.when(g + 1 < n_gr)
                def _():
                    idx_copy(c0 + _NBUF + b, b).start()
            return carry

        lax.fori_loop(0, n_gr, group, 0)
        for b in range(_NBUF):
            out_copy(0, b).wait()

    return gk(table, idx)


def _sc_gather2(table_a, idx_a, table_b, idx_b):
    d = table_a.shape[1]
    e = idx_a.shape[0]
    per_w = e // _NW
    n_ch = per_w // _CW
    n_gr = n_ch // _NBUF
    mesh = plsc.VectorSubcoreMesh(core_axis_name="c", subcore_axis_name="s")

    @functools.partial(
        pl.kernel,
        out_type=(jax.ShapeDtypeStruct((e, d), jnp.float32),
                  jax.ShapeDtypeStruct((e, d), jnp.float32)),
        mesh=mesh,
        scratch_types=[
            pltpu.VMEM((_NBUF, _CW), jnp.int32),
            pltpu.VMEM((_NBUF, _CW), jnp.int32),
            pltpu.VMEM((_NBUF, _CW, d), jnp.float32),
            pltpu.VMEM((_NBUF, _CW, d), jnp.float32),
            pltpu.SemaphoreType.DMA((_NBUF,)),
            pltpu.SemaphoreType.DMA((_NBUF,)),
            pltpu.SemaphoreType.DMA((_NBUF,)),
            pltpu.SemaphoreType.DMA((_NBUF,)),
            pltpu.SemaphoreType.DMA((_NBUF,)),
            pltpu.SemaphoreType.DMA((_NBUF,)),
        ],
    )
    def gk(ta_hbm, ia_hbm, tb_hbm, ib_hbm, outa_hbm, outb_hbm,
           ia_v, ib_v, ra_v, rb_v, isema, isemb, gsema, gsemb, ssema, ssemb):
        wid = lax.axis_index("s") * _NC + lax.axis_index("c")
        base_w = wid * per_w

        def icopy(hbm, vref, sem, c, b):
            return pltpu.make_async_copy(
                hbm.at[pl.ds(base_w + c * _CW, _CW)], vref.at[b], sem.at[b])

        def ocopy(vref, hbm, sem, c, b):
            return pltpu.make_async_copy(
                vref.at[b], hbm.at[pl.ds(base_w + c * _CW, _CW)], sem.at[b])

        for b in range(_NBUF):
            icopy(ia_hbm, ia_v, isema, b, b).start()
            icopy(ib_hbm, ib_v, isemb, b, b).start()

        def group(g, carry):
            c0 = g * _NBUF
            gathers = []
            for b in range(_NBUF):
                @pl.when(g > 0)
                def _():
                    ocopy(ra_v, outa_hbm, ssema, 0, b).wait()
                    ocopy(rb_v, outb_hbm, ssemb, 0, b).wait()

                icopy(ia_hbm, ia_v, isema, c0 + b, b).wait()
                icopy(ib_hbm, ib_v, isemb, c0 + b, b).wait()
                gathers.append((
                    pltpu.async_copy(ta_hbm.at[ia_v.at[b]], ra_v.at[b],
                                     gsema.at[b]),
                    pltpu.async_copy(tb_hbm.at[ib_v.at[b]], rb_v.at[b],
                                     gsemb.at[b])))
            for b in range(_NBUF):
                ga, gb = gathers[b]
                ga.wait()
                ocopy(ra_v, outa_hbm, ssema, c0 + b, b).start()
                gb.wait()
                ocopy(rb_v, outb_hbm, ssemb, c0 + b, b).start()

                @pl.when(g + 1 < n_gr)
                def _():
                    icopy(ia_hbm, ia_v, isema, c0 + _NBUF + b, b).start()
                    icopy(ib_hbm, ib_v, isemb, c0 + _NBUF + b, b).start()
            return carry

        lax.fori_loop(0, n_gr, group, 0)
        for b in range(_NBUF):
            ocopy(ra_v, outa_hbm, ssema, 0, b).wait()
            ocopy(rb_v, outb_hbm, ssemb, 0, b).wait()

    return gk(table_a, idx_a, table_b, idx_b)


def _sc_scatter_add(vals, idx, zeros_nd):
    e, d = vals.shape
    n = zeros_nd.shape[0]
    per_w = e // _NW
    cw = 40
    n_ch = per_w // cw
    rpt = 640
    tail = n - (_NS - 1) * rpt
    mesh = plsc.VectorSubcoreMesh(core_axis_name="c", subcore_axis_name="s")

    n_gr = n_ch // _NBUF

    @functools.partial(
        pl.kernel,
        out_type=jax.ShapeDtypeStruct((_NC, n, d), jnp.float32),
        mesh=mesh,
        scratch_types=[
            pltpu.VMEM((_NBUF, cw), jnp.int32),
            pltpu.VMEM((_NBUF, cw, d), jnp.float32),
            pltpu.VMEM_SHARED((n, d), jnp.float32),
            pltpu.SemaphoreType.DMA((_NBUF,)),
            pltpu.SemaphoreType.DMA((_NBUF,)),
            pltpu.SemaphoreType.DMA((_NBUF,)),
        ],
    )
    def sk(vals_hbm, idx_hbm, zeros_hbm, out_hbm, idx_v, rows_v, acc_sh,
           isem, vsem, ssem):
        cid = lax.axis_index("c")
        sid = lax.axis_index("s")
        wid = sid * _NC + cid
        base_w = wid * per_w

        @pl.when(sid < _NS - 1)
        def _():
            pltpu.sync_copy(zeros_hbm.at[pl.ds(sid * rpt, rpt)],
                            acc_sh.at[pl.ds(sid * rpt, rpt)])

        @pl.when(sid == _NS - 1)
        def _():
            pltpu.sync_copy(zeros_hbm.at[pl.ds((_NS - 1) * rpt, tail)],
                            acc_sh.at[pl.ds((_NS - 1) * rpt, tail)])

        plsc.subcore_barrier()

        def idx_copy(c, b):
            return pltpu.make_async_copy(
                idx_hbm.at[pl.ds(base_w + c * cw, cw)], idx_v.at[b],
                isem.at[b])

        def val_copy(c, b):
            return pltpu.make_async_copy(
                vals_hbm.at[pl.ds(base_w + c * cw, cw)], rows_v.at[b],
                vsem.at[b])

        for b in range(_NBUF):
            idx_copy(b, b).start()
            val_copy(b, b).start()

        def group(g, carry):
            c0 = g * _NBUF
            scat = []
            for b in range(_NBUF):
                idx_copy(c0 + b, b).wait()
                val_copy(c0 + b, b).wait()
                scat.append(pltpu.async_copy(
                    rows_v.at[b], acc_sh.at[idx_v.at[b]], ssem.at[b],
                    add=True))
            for b in range(_NBUF):
                scat[b].wait()

                @pl.when(g + 1 < n_gr)
                def _():
                    idx_copy(c0 + _NBUF + b, b).start()
                    val_copy(c0 + _NBUF + b, b).start()
            return carry

        lax.fori_loop(0, n_gr, group, 0)
        plsc.subcore_barrier()

        @pl.when(sid < _NS - 1)
        def _():
            pltpu.sync_copy(acc_sh.at[pl.ds(sid * rpt, rpt)],
                            out_hbm.at[cid, pl.ds(sid * rpt, rpt)])

        @pl.when(sid == _NS - 1)
        def _():
            pltpu.sync_copy(acc_sh.at[pl.ds((_NS - 1) * rpt, tail)],
                            out_hbm.at[cid, pl.ds((_NS - 1) * rpt, tail)])

    return sk(vals, idx, zeros_nd)


def _tc_matmul(a, w):
    m, k = a.shape
    n = w.shape[1]
    bs = 2000

    def body(a_ref, w_ref, o_ref):
        o_ref[...] = jnp.dot(a_ref[...], w_ref[...],
                             preferred_element_type=jnp.float32)

    return pl.pallas_call(
        body,
        grid=(m // bs,),
        in_specs=[_row_block(bs, k), _rep_block((k, n))],
        out_specs=_row_block(bs, n),
        out_shape=jax.ShapeDtypeStruct((m, n), jnp.float32),
    )(a, w)


def _tc_init(g, ea, wie, bi, wh):
    e, h = g.shape
    de = ea.shape[1]
    bs = 2560

    def body(g_ref, ea_ref, wie_ref, bi_ref, wh_ref, h0_ref, p_ref):
        h0 = (g_ref[...]
              + jnp.dot(ea_ref[...], wie_ref[...],
                        preferred_element_type=jnp.float32)
              + bi_ref[...])
        h0_ref[...] = h0
        p_ref[...] = jnp.dot(jnp.maximum(h0, 0.0), wh_ref[...],
                             preferred_element_type=jnp.float32)

    return pl.pallas_call(
        body,
        grid=(e // bs,),
        in_specs=[_row_block(bs, h), _row_block(bs, de), _rep_block((de, h)),
                  _rep_block((1, h)), _rep_block((h, h))],
        out_specs=[_row_block(bs, h), _row_block(bs, h)],
        out_shape=[jax.ShapeDtypeStruct((e, h), jnp.float32),
                   jax.ShapeDtypeStruct((e, h), jnp.float32)],
    )(g, ea, wie, bi, wh)


def _tc_step(h0, a, b, bh, wh):
    e, h = h0.shape
    bs = 2560

    def body(h0_ref, a_ref, b_ref, bh_ref, wh_ref, p_ref):
        ht = jnp.maximum(h0_ref[...] + a_ref[...] - b_ref[...] + bh_ref[...],
                         0.0)
        p_ref[...] = jnp.dot(ht, wh_ref[...],
                             preferred_element_type=jnp.float32)

    return pl.pallas_call(
        body,
        grid=(e // bs,),
        in_specs=[_row_block(bs, h)] * 3 + [_rep_block((1, h)),
                                            _rep_block((h, h))],
        out_specs=_row_block(bs, h),
        out_shape=jax.ShapeDtypeStruct((e, h), jnp.float32),
    )(h0, a, b, bh, wh)


def _tc_last(h0, a, b, bh):
    e, h = h0.shape
    bs = 2560

    def body(h0_ref, a_ref, b_ref, bh_ref, o_ref):
        o_ref[...] = jnp.maximum(
            h0_ref[...] + a_ref[...] - b_ref[...] + bh_ref[...], 0.0)

    return pl.pallas_call(
        body,
        grid=(e // bs,),
        in_specs=[_row_block(bs, h)] * 3 + [_rep_block((1, h))],
        out_specs=_row_block(bs, h),
        out_shape=jax.ShapeDtypeStruct((e, h), jnp.float32),
    )(h0, a, b, bh)


def _tc_merge(p0, p1):
    n, h = p0.shape
    bs = 2000

    def body(a_ref, b_ref, o_ref):
        o_ref[...] = a_ref[...] + b_ref[...]

    return pl.pallas_call(
        body,
        grid=(n // bs,),
        in_specs=[_row_block(bs, h)] * 2,
        out_specs=_row_block(bs, h),
        out_shape=jax.ShapeDtypeStruct((n, h), jnp.float32),
    )(p0, p1)


def _tc_final(p0, p1, x, wox, wom, bo):
    n, h = x.shape
    bs = 2000

    def body(p0_ref, p1_ref, x_ref, wox_ref, wom_ref, bo_ref, o_ref):
        ns = p0_ref[...] + p1_ref[...]
        s = jnp.sum(ns, axis=1, keepdims=True)
        m = jnp.where(s == 0.0, x_ref[...], ns)
        o_ref[...] = jnp.maximum(
            jnp.dot(x_ref[...], wox_ref[...],
                    preferred_element_type=jnp.float32)
            + jnp.dot(m, wom_ref[...], preferred_element_type=jnp.float32)
            + bo_ref[...], 0.0)

    return pl.pallas_call(
        body,
        grid=(n // bs,),
        in_specs=[_row_block(bs, h)] * 3 + [_rep_block((h, h))] * 2
                 + [_rep_block((1, h))],
        out_specs=_row_block(bs, h),
        out_shape=jax.ShapeDtypeStruct((n, h), jnp.float32),
    )(p0, p1, x, wox, wom, bo)


def kernel(x, edge_index, rev_edge_index, edge_attr, W_i, b_i, W_h, b_h,
           W_o, b_o):
    n, df = x.shape
    h = W_h.shape[0]
    src = edge_index[0]
    dst = edge_index[1]
    wi_x, wi_e = W_i[:df], W_i[df:]
    wo_x, wo_m = W_o[:df], W_o[df:]
    bi = b_i.reshape(1, h)
    bh = b_h.reshape(1, h)
    bo = b_o.reshape(1, h)
    zeros_nd = jnp.zeros((n, h), jnp.float32)

    hx = _tc_matmul(x, wi_x)
    g0 = _sc_gather(hx, src)
    h0, p = _tc_init(g0, edge_attr, wi_e, bi, W_h)

    for _ in range(_DEPTH - 2):
        parts = _sc_scatter_add(p, dst, zeros_nd)
        ns = _tc_merge(parts[0], parts[1])
        a, b = _sc_gather2(ns, src, p, rev_edge_index)
        p = _tc_step(h0, a, b, bh, W_h)

    parts = _sc_scatter_add(p, dst, zeros_nd)
    ns = _tc_merge(parts[0], parts[1])
    a, b = _sc_gather2(ns, src, p, rev_edge_index)
    h_fin = _tc_last(h0, a, b, bh)

    parts = _sc_scatter_add(h_fin, dst, zeros_nd)
    return _tc_final(parts[0], parts[1], x, wo_x, wo_m, bo)
